# baseline (speedup 1.0000x reference)
import functools

import jax
import jax.numpy as jnp
from jax import lax
from jax.experimental import pallas as pl
from jax.experimental.pallas import tpu as pltpu
from jax.experimental.pallas import tpu_sc as plsc

VOCAB = 148
P = 50
D = 128
B = 4096
VPAD = 160
NC, NS = 2, 16
NW = NC * NS
LANES = 16

B_SC = 1024
BPW = B_SC // NW
CHUNK = 2
NCHUNK = 25
POS_ROWS = NCHUNK * CHUNK
ZROW = VPAD + VOCAB


def _proj_body(table_ref, w_ref, b_ref, out_ref):
    p = pl.program_id(0)
    acc = jnp.dot(table_ref[...], w_ref[0],
                  preferred_element_type=jnp.float32)
    scale = jnp.where(p == 0, 1.0, 0.0).astype(jnp.float32)
    out_ref[0] = acc + scale * b_ref[0]


def _build_m(table_pad, w3, bias_row):
    return pl.pallas_call(
        _proj_body,
        grid=(P,),
        in_specs=[
            pl.BlockSpec((VPAD, D), lambda p: (0, 0)),
            pl.BlockSpec((1, D, D), lambda p: (p, 0, 0)),
            pl.BlockSpec((1, D), lambda p: (0, 0)),
        ],
        out_specs=pl.BlockSpec((1, VPAD, D), lambda p: (p, 0, 0)),
        out_shape=jax.ShapeDtypeStruct((P, VPAD, D), jnp.float32),
    )(table_pad, w3, bias_row)


_mesh = plsc.VectorSubcoreMesh(core_axis_name="c", subcore_axis_name="s")


@functools.partial(
    pl.kernel,
    mesh=_mesh,
    out_type=jax.ShapeDtypeStruct((B_SC, D), jnp.float32),
    scratch_types=[
        pltpu.VMEM((POS_ROWS * BPW,), jnp.int32),
        pltpu.VMEM((CHUNK * BPW, D), jnp.float32),
        pltpu.VMEM((CHUNK * BPW, D), jnp.float32),
        pltpu.VMEM((BPW, D), jnp.float32),
        pltpu.SemaphoreType.DMA,
        pltpu.SemaphoreType.DMA,
    ],
)
def _sc_gather_sum(m_hbm, idx_hbm, out_hbm, idx_v, buf_a, buf_b, acc_v,
                   ga, gb):
    c = lax.axis_index("c")
    s = lax.axis_index("s")
    wid = s * NC + c
    cb = CHUNK * BPW

    pltpu.sync_copy(idx_hbm.at[wid], idx_v)

    def gidx(t):
        return idx_v.at[pl.ds(cb * t, cb)]

    zero = jnp.zeros((LANES,), jnp.float32)

    def zero_body(i, carry):
        for k in range(D // LANES):
            acc_v[i, pl.ds(k * LANES, LANES)] = zero
        return carry

    lax.fori_loop(0, BPW, zero_body, 0)

    def accum(buf):
        def row_body(i, carry):
            for k in range(D // LANES):
                sl = pl.ds(k * LANES, LANES)
                v = buf[i, sl]
                for r in range(1, CHUNK):
                    v = v + buf[r * BPW + i, sl]
                plsc.addupdate(acc_v.at[i, sl], v)
            return carry

        lax.fori_loop(0, BPW, row_body, 0)

    pltpu.async_copy(m_hbm.at[gidx(0)], buf_a, ga)

    def pair_body(t, carry):
        u = 2 * t
        pltpu.make_async_copy(m_hbm.at[gidx(u)], buf_a, ga).wait()
        pltpu.async_copy(m_hbm.at[gidx(u + 1)], buf_b, gb)
        accum(buf_a)
        pltpu.make_async_copy(m_hbm.at[gidx(u + 1)], buf_b, gb).wait()
        pltpu.async_copy(m_hbm.at[gidx(u + 2)], buf_a, ga)
        accum(buf_b)
        return carry

    lax.fori_loop(0, (NCHUNK - 1) // 2, pair_body, 0)
    pltpu.make_async_copy(m_hbm.at[gidx(NCHUNK - 1)], buf_a, ga).wait()
    accum(buf_a)

    pltpu.sync_copy(acc_v, out_hbm.at[pl.ds(wid * BPW, BPW)])


BT = 3072


def _tc_body(x_ref, m_ref, out_ref):
    p = pl.program_id(1)
    xrow = x_ref[0, 0, :]
    iota = lax.broadcasted_iota(jnp.int32, (VPAD, BT), 0)
    oh = (iota == xrow[None, :]).astype(jnp.bfloat16)
    partial = lax.dot_general(oh, m_ref[0], (((0,), (0,)), ((), ())),
                              preferred_element_type=jnp.float32)

    @pl.when(p == 0)
    def _():
        out_ref[...] = partial

    @pl.when(p != 0)
    def _():
        out_ref[...] += partial


def _tc_onehot_sum(x3, m3, n_rows):
    return pl.pallas_call(
        _tc_body,
        grid=(n_rows // BT, P),
        in_specs=[
            pl.BlockSpec((1, 1, BT), lambda bt, p: (p, 0, bt)),
            pl.BlockSpec((1, VPAD, D), lambda bt, p: (p, 0, 0)),
        ],
        out_specs=pl.BlockSpec((BT, D), lambda bt, p: (bt, 0)),
        out_shape=jax.ShapeDtypeStruct((n_rows, D), jnp.float32),
    )(x3, m3)


def kernel(x, table, W, b):
    table_pad = jnp.zeros((VPAD, D), jnp.float32).at[:VOCAB].set(table)
    w3 = W.reshape(P, D, D)
    m = _build_m(table_pad, w3, b.reshape(1, D)).reshape(P * VPAD, D)

    xi = x.astype(jnp.int32)

    xw = xi[:B_SC].reshape(NW, BPW, P).transpose(0, 2, 1)
    fidx = xw + (jnp.arange(P, dtype=jnp.int32) * VPAD)[None, :, None]
    pad = jnp.full((NW, POS_ROWS - P, BPW), ZROW, jnp.int32)
    fidx = jnp.concatenate([fidx, pad], axis=1).reshape(NW, -1)
    out_sc = _sc_gather_sum(m, fidx)

    x3 = xi[B_SC:].transpose(1, 0)[:, None, :]
    m3 = m.astype(jnp.bfloat16).reshape(P, VPAD, D)
    out_tc = _tc_onehot_sum(x3, m3, B - B_SC)

    return jnp.concatenate([out_sc, out_tc], axis=0)


# speedup vs baseline: 1.3243x; 1.3243x over previous
import functools

import jax
import jax.numpy as jnp
from jax import lax
from jax.experimental import pallas as pl
from jax.experimental.pallas import tpu as pltpu
from jax.experimental.pallas import tpu_sc as plsc

VOCAB = 148
P = 50
D = 128
B = 4096
VPAD = 160
NC, NS = 2, 16
NW = NC * NS
LANES = 16

B_SC = 2048
BPW = B_SC // NW
CHUNK = 2
NCHUNK = 25
POS_ROWS = NCHUNK * CHUNK
ZROW = VPAD + VOCAB


PBLK = 5


def _proj_body(table_ref, w_ref, b_ref, out_ref, out16_ref):
    pid = pl.program_id(0)
    for q in range(PBLK):
        acc = jnp.dot(table_ref[...], w_ref[q],
                      preferred_element_type=jnp.float32)
        if q == 0:
            scale = jnp.where(pid == 0, 1.0, 0.0).astype(jnp.float32)
            acc = acc + scale * b_ref[0]
        out_ref[q] = acc
        out16_ref[q] = acc.astype(jnp.bfloat16)


def _build_m(table_pad, w3, bias_row):
    return pl.pallas_call(
        _proj_body,
        grid=(P // PBLK,),
        in_specs=[
            pl.BlockSpec((VPAD, D), lambda p: (0, 0)),
            pl.BlockSpec((PBLK, D, D), lambda p: (p, 0, 0)),
            pl.BlockSpec((1, D), lambda p: (0, 0)),
        ],
        out_specs=[
            pl.BlockSpec((PBLK, VPAD, D), lambda p: (p, 0, 0)),
            pl.BlockSpec((PBLK, VPAD, D), lambda p: (p, 0, 0)),
        ],
        out_shape=[
            jax.ShapeDtypeStruct((P, VPAD, D), jnp.float32),
            jax.ShapeDtypeStruct((P, VPAD, D), jnp.bfloat16),
        ],
    )(table_pad, w3, bias_row)


_mesh = plsc.VectorSubcoreMesh(core_axis_name="c", subcore_axis_name="s")


@functools.partial(
    pl.kernel,
    mesh=_mesh,
    out_type=jax.ShapeDtypeStruct((B_SC, D), jnp.float32),
    scratch_types=[
        pltpu.VMEM((POS_ROWS * BPW,), jnp.int32),
        pltpu.VMEM((CHUNK * BPW, D), jnp.float32),
        pltpu.VMEM((CHUNK * BPW, D), jnp.float32),
        pltpu.VMEM((BPW, D), jnp.float32),
        pltpu.SemaphoreType.DMA,
        pltpu.SemaphoreType.DMA,
    ],
)
def _sc_gather_sum(m_hbm, idx_hbm, out_hbm, idx_v, buf_a, buf_b, acc_v,
                   ga, gb):
    c = lax.axis_index("c")
    s = lax.axis_index("s")
    wid = s * NC + c
    cb = CHUNK * BPW

    pltpu.sync_copy(idx_hbm.at[wid], idx_v)

    def gidx(t):
        return idx_v.at[pl.ds(cb * t, cb)]

    zero = jnp.zeros((LANES,), jnp.float32)

    def zero_body(i, carry):
        for k in range(D // LANES):
            acc_v[i, pl.ds(k * LANES, LANES)] = zero
        return carry

    lax.fori_loop(0, BPW, zero_body, 0)

    def accum(buf):
        def row_body(i, carry):
            for k in range(D // LANES):
                sl = pl.ds(k * LANES, LANES)
                v = buf[i, sl]
                for r in range(1, CHUNK):
                    v = v + buf[r * BPW + i, sl]
                plsc.addupdate(acc_v.at[i, sl], v)
            return carry

        lax.fori_loop(0, BPW, row_body, 0)

    pltpu.async_copy(m_hbm.at[gidx(0)], buf_a, ga)

    def pair_body(t, carry):
        u = 2 * t
        pltpu.make_async_copy(m_hbm.at[gidx(u)], buf_a, ga).wait()
        pltpu.async_copy(m_hbm.at[gidx(u + 1)], buf_b, gb)
        accum(buf_a)
        pltpu.make_async_copy(m_hbm.at[gidx(u + 1)], buf_b, gb).wait()
        pltpu.async_copy(m_hbm.at[gidx(u + 2)], buf_a, ga)
        accum(buf_b)
        return carry

    lax.fori_loop(0, (NCHUNK - 1) // 2, pair_body, 0)
    pltpu.make_async_copy(m_hbm.at[gidx(NCHUNK - 1)], buf_a, ga).wait()
    accum(buf_a)

    pltpu.sync_copy(acc_v, out_hbm.at[pl.ds(wid * BPW, BPW)])


BT = 2048


def _tc_body(x_ref, m_ref, out_ref):
    p = pl.program_id(1)
    xrow = x_ref[0, 0, :]
    iota = lax.broadcasted_iota(jnp.int32, (VPAD, BT), 0)
    oh = (iota == xrow[None, :]).astype(jnp.bfloat16)
    partial = lax.dot_general(oh, m_ref[0], (((0,), (0,)), ((), ())),
                              preferred_element_type=jnp.float32)

    @pl.when(p == 0)
    def _():
        out_ref[...] = partial

    @pl.when(p != 0)
    def _():
        out_ref[...] += partial


def _tc_onehot_sum(x3, m3, n_rows):
    return pl.pallas_call(
        _tc_body,
        grid=(n_rows // BT, P),
        in_specs=[
            pl.BlockSpec((1, 1, BT), lambda bt, p: (p, 0, bt)),
            pl.BlockSpec((1, VPAD, D), lambda bt, p: (p, 0, 0)),
        ],
        out_specs=pl.BlockSpec((BT, D), lambda bt, p: (bt, 0)),
        out_shape=jax.ShapeDtypeStruct((n_rows, D), jnp.float32),
    )(x3, m3)


def kernel(x, table, W, b):
    table_pad = jnp.zeros((VPAD, D), jnp.float32).at[:VOCAB].set(table)
    w3 = W.reshape(P, D, D)
    m32, m3 = _build_m(table_pad, w3, b.reshape(1, D))
    m = m32.reshape(P * VPAD, D)

    xi = x.astype(jnp.int32)

    xw = xi[:B_SC].reshape(NW, BPW, P).transpose(0, 2, 1)
    fidx = xw + (jnp.arange(P, dtype=jnp.int32) * VPAD)[None, :, None]
    pad = jnp.full((NW, POS_ROWS - P, BPW), ZROW, jnp.int32)
    fidx = jnp.concatenate([fidx, pad], axis=1).reshape(NW, -1)
    out_sc = _sc_gather_sum(m, fidx)

    x3 = xi[B_SC:].transpose(1, 0)[:, None, :]
    out_tc = _tc_onehot_sum(x3, m3, B - B_SC)

    return jnp.concatenate([out_sc, out_tc], axis=0)


# speedup vs baseline: 1.3572x; 1.0248x over previous
import functools

import jax
import jax.numpy as jnp
from jax import lax
from jax.experimental import pallas as pl
from jax.experimental.pallas import tpu as pltpu
from jax.experimental.pallas import tpu_sc as plsc

VOCAB = 148
P = 50
D = 128
B = 4096
VPAD = 160
NC, NS = 2, 16
NW = NC * NS
LANES = 16

B_SC = 2048
BPW = B_SC // NW
CHUNK = 2
NCHUNK = 25
POS_ROWS = NCHUNK * CHUNK
ZROW = VPAD + VOCAB


PBLK = 10


def _proj_body(table_ref, w_ref, b_ref, out_ref, out16_ref):
    pid = pl.program_id(0)
    for q in range(PBLK):
        acc = jnp.dot(table_ref[...], w_ref[q],
                      preferred_element_type=jnp.float32)
        if q == 0:
            scale = jnp.where(pid == 0, 1.0, 0.0).astype(jnp.float32)
            acc = acc + scale * b_ref[0]
        out_ref[q] = acc
        out16_ref[q] = acc.astype(jnp.bfloat16)


def _build_m(table_pad, w3, bias_row):
    return pl.pallas_call(
        _proj_body,
        grid=(P // PBLK,),
        in_specs=[
            pl.BlockSpec((VPAD, D), lambda p: (0, 0)),
            pl.BlockSpec((PBLK, D, D), lambda p: (p, 0, 0)),
            pl.BlockSpec((1, D), lambda p: (0, 0)),
        ],
        out_specs=[
            pl.BlockSpec((PBLK, VPAD, D), lambda p: (p, 0, 0)),
            pl.BlockSpec((PBLK, VPAD, D), lambda p: (p, 0, 0)),
        ],
        out_shape=[
            jax.ShapeDtypeStruct((P, VPAD, D), jnp.float32),
            jax.ShapeDtypeStruct((P, VPAD, D), jnp.bfloat16),
        ],
    )(table_pad, w3, bias_row)


_mesh = plsc.VectorSubcoreMesh(core_axis_name="c", subcore_axis_name="s")


@functools.partial(
    pl.kernel,
    mesh=_mesh,
    out_type=jax.ShapeDtypeStruct((B_SC, D), jnp.float32),
    scratch_types=[
        pltpu.VMEM((POS_ROWS * BPW,), jnp.int32),
        pltpu.VMEM((CHUNK * BPW, D), jnp.float32),
        pltpu.VMEM((CHUNK * BPW, D), jnp.float32),
        pltpu.VMEM((BPW, D), jnp.float32),
        pltpu.SemaphoreType.DMA,
        pltpu.SemaphoreType.DMA,
    ],
)
def _sc_gather_sum(m_hbm, idx_hbm, out_hbm, idx_v, buf_a, buf_b, acc_v,
                   ga, gb):
    c = lax.axis_index("c")
    s = lax.axis_index("s")
    wid = s * NC + c
    cb = CHUNK * BPW

    pltpu.sync_copy(idx_hbm.at[wid], idx_v)

    def gidx(t):
        return idx_v.at[pl.ds(cb * t, cb)]

    zero = jnp.zeros((LANES,), jnp.float32)

    def zero_body(i, carry):
        for k in range(D // LANES):
            acc_v[i, pl.ds(k * LANES, LANES)] = zero
        return carry

    lax.fori_loop(0, BPW, zero_body, 0)

    def accum(buf):
        def row_body(i, carry):
            for k in range(D // LANES):
                sl = pl.ds(k * LANES, LANES)
                v = buf[i, sl]
                for r in range(1, CHUNK):
                    v = v + buf[r * BPW + i, sl]
                plsc.addupdate(acc_v.at[i, sl], v)
            return carry

        lax.fori_loop(0, BPW, row_body, 0)

    pltpu.async_copy(m_hbm.at[gidx(0)], buf_a, ga)

    def pair_body(t, carry):
        u = 2 * t
        pltpu.make_async_copy(m_hbm.at[gidx(u)], buf_a, ga).wait()
        pltpu.async_copy(m_hbm.at[gidx(u + 1)], buf_b, gb)
        accum(buf_a)
        pltpu.make_async_copy(m_hbm.at[gidx(u + 1)], buf_b, gb).wait()
        pltpu.async_copy(m_hbm.at[gidx(u + 2)], buf_a, ga)
        accum(buf_b)
        return carry

    lax.fori_loop(0, (NCHUNK - 1) // 2, pair_body, 0)
    pltpu.make_async_copy(m_hbm.at[gidx(NCHUNK - 1)], buf_a, ga).wait()
    accum(buf_a)

    pltpu.sync_copy(acc_v, out_hbm.at[pl.ds(wid * BPW, BPW)])


BT = 2048


def _tc_body(x_ref, m_ref, out_ref):
    p = pl.program_id(1)
    xrow = x_ref[0, 0, :]
    iota = lax.broadcasted_iota(jnp.int32, (VPAD, BT), 0)
    oh = (iota == xrow[None, :]).astype(jnp.bfloat16)
    partial = lax.dot_general(oh, m_ref[0], (((0,), (0,)), ((), ())),
                              preferred_element_type=jnp.float32)

    @pl.when(p == 0)
    def _():
        out_ref[...] = partial

    @pl.when(p != 0)
    def _():
        out_ref[...] += partial


def _tc_onehot_sum(x3, m3, n_rows):
    return pl.pallas_call(
        _tc_body,
        grid=(n_rows // BT, P),
        in_specs=[
            pl.BlockSpec((1, 1, BT), lambda bt, p: (p, 0, bt)),
            pl.BlockSpec((1, VPAD, D), lambda bt, p: (p, 0, 0)),
        ],
        out_specs=pl.BlockSpec((BT, D), lambda bt, p: (bt, 0)),
        out_shape=jax.ShapeDtypeStruct((n_rows, D), jnp.float32),
    )(x3, m3)


def kernel(x, table, W, b):
    table_pad = jnp.zeros((VPAD, D), jnp.float32).at[:VOCAB].set(table)
    w3 = W.reshape(P, D, D)
    m32, m3 = _build_m(table_pad, w3, b.reshape(1, D))
    m = m32.reshape(P * VPAD, D)

    xi = x.astype(jnp.int32)

    xw = xi[:B_SC].reshape(NW, BPW, P).transpose(0, 2, 1)
    fidx = xw + (jnp.arange(P, dtype=jnp.int32) * VPAD)[None, :, None]
    pad = jnp.full((NW, POS_ROWS - P, BPW), ZROW, jnp.int32)
    fidx = jnp.concatenate([fidx, pad], axis=1).reshape(NW, -1)
    out_sc = _sc_gather_sum(m, fidx)

    x3 = xi[B_SC:].transpose(1, 0)[:, None, :]
    out_tc = _tc_onehot_sum(x3, m3, B - B_SC)

    return jnp.concatenate([out_sc, out_tc], axis=0)
